# baseline (speedup 1.0000x reference)
import functools

import jax
import jax.numpy as jnp
from jax import lax
from jax.experimental import pallas as pl
from jax.experimental.pallas import tpu as pltpu
from jax.experimental.pallas import tpu_sc as plsc

_F = 128
_RES = 512
_RP = _RES + 1
_FS = _RES * _RP
_HALF = _F * _FS // 2
_N = 65536
_NC, _NS, _L = 2, 16, 16
_NW = _NC * _NS
_ROWS_W = _N // _NW
_CROWS = 32
_NCHUNK = _ROWS_W // _CROWS
_VPR = _F // _L
_CE = _CROWS * _F

_MESH = plsc.VectorSubcoreMesh(core_axis_name="c", subcore_axis_name="s")

_f32 = jnp.float32
_i32 = jnp.int32


@functools.partial(
    pl.kernel,
    out_type=jax.ShapeDtypeStruct((_N, _F), _f32),
    mesh=_MESH,
    compiler_params=pltpu.CompilerParams(use_tc_tiling_on_sc=False,
                                         needs_layout_passes=False),
    scratch_types=[
        pltpu.VMEM((_CROWS, _F), _f32),
        pltpu.VMEM((_CROWS, _F), _f32),
        pltpu.VMEM((_CE,), _i32),
        pltpu.VMEM((_CE,), _i32),
        pltpu.VMEM((_CE, 2), _f32),
        pltpu.VMEM((_CE, 2), _f32),
        pltpu.VMEM((_CE,), _f32),
        pltpu.VMEM((_CE,), _f32),
        pltpu.VMEM((_CROWS, _F), _f32),
        pltpu.SemaphoreType.DMA,
    ],
)
def _sc_sample(x_hbm, y_hbm, tab_hbm, out_hbm,
               xv, yv, idx0, idx1, rbuf0, rbuf1, xwv, ywv, outv, sem):
    wid = lax.axis_index("s") * _NC + lax.axis_index("c")
    row0 = wid * _ROWS_W

    lane = lax.iota(_i32, _L)
    zeros16 = jnp.zeros((_L,), _i32)
    ones16 = jnp.ones((_L,), _i32)

    def chunk_body(k, _):
        rbase = row0 + k * _CROWS
        pltpu.sync_copy(x_hbm.at[pl.ds(rbase, _CROWS)], xv)
        pltpu.sync_copy(y_hbm.at[pl.ds(rbase, _CROWS)], yv)

        def idx_row(j, _):
            for c in range(_VPR):
                sl = pl.ds(c * _L, _L)
                fl = pl.ds(j * _F + c * _L, _L)
                x = xv[j, sl] - 0.5
                y = yv[j, sl] - 0.5
                xi = x.astype(_i32)
                yi = y.astype(_i32)
                x0 = jnp.where(xi.astype(_f32) > x, xi - 1, xi)
                y0 = jnp.where(yi.astype(_f32) > y, yi - 1, yi)
                xwv[fl] = x - x0.astype(_f32)
                ywv[fl] = y - y0.astype(_f32)
                x0m = jnp.bitwise_and(x0, _RES - 1)
                y0s = jnp.bitwise_and(y0, _RES - 1) * _RP
                y1s = jnp.bitwise_and(y0 + 1, _RES - 1) * _RP
                fbase = (lane + c * _L) * _FS
                o0 = fbase + y0s + x0m
                o1 = fbase + y1s + x0m
                idx0[fl] = (lax.shift_right_logical(o0, 1)
                            + jnp.bitwise_and(o0, 1) * _HALF)
                idx1[fl] = (lax.shift_right_logical(o1, 1)
                            + jnp.bitwise_and(o1, 1) * _HALF)
            return 0

        lax.fori_loop(0, _CROWS, idx_row, 0)

        pltpu.async_copy(tab_hbm.at[idx0], rbuf0, sem)
        pltpu.async_copy(tab_hbm.at[idx1], rbuf1, sem)
        pltpu.make_async_copy(tab_hbm.at[idx0], rbuf0, sem).wait()
        pltpu.make_async_copy(tab_hbm.at[idx1], rbuf1, sem).wait()

        def lerp_row(j, _):
            for c in range(_VPR):
                sl = pl.ds(c * _L, _L)
                fl = pl.ds(j * _F + c * _L, _L)
                ev = j * _F + c * _L + lane
                a00 = plsc.load_gather(rbuf0, [ev, zeros16])
                a01 = plsc.load_gather(rbuf0, [ev, ones16])
                a10 = plsc.load_gather(rbuf1, [ev, zeros16])
                a11 = plsc.load_gather(rbuf1, [ev, ones16])
                xw = xwv[fl]
                yw = ywv[fl]
                i0 = a00 + (a01 - a00) * xw
                i1 = a10 + (a11 - a10) * xw
                outv[j, sl] = i0 + (i1 - i0) * yw
            return 0

        lax.fori_loop(0, _CROWS, lerp_row, 0)

        pltpu.sync_copy(outv, out_hbm.at[pl.ds(rbase, _CROWS)])
        return 0

    lax.fori_loop(0, _NCHUNK, chunk_body, 0)


def kernel(coords, image):
    x = coords[:, :, 0]
    y = coords[:, :, 1]
    flatp = jnp.concatenate([image, image[:, :, :1]], axis=2).reshape(-1)
    rolled = jnp.concatenate([flatp[1:], flatp[:1]])
    tab = jnp.concatenate([flatp, rolled]).reshape(-1, 2)
    return _sc_sample(x, y, tab)


# speedup vs baseline: 1.0511x; 1.0511x over previous
import functools

import jax
import jax.numpy as jnp
from jax import lax
from jax.experimental import pallas as pl
from jax.experimental.pallas import tpu as pltpu
from jax.experimental.pallas import tpu_sc as plsc

_F = 128
_RES = 512
_RP = _RES
_FS = _RES * _RP
_HALF = _F * _FS // 2
_N = 65536
_NC, _NS, _L = 2, 16, 16
_NW = _NC * _NS
_ROWS_W = _N // _NW
_CROWS = 32
_NCHUNK = _ROWS_W // _CROWS
_VPR = _F // _L
_CE = _CROWS * _F

_MESH = plsc.VectorSubcoreMesh(core_axis_name="c", subcore_axis_name="s")

_f32 = jnp.float32
_i32 = jnp.int32


@functools.partial(
    pl.kernel,
    out_type=jax.ShapeDtypeStruct((_N, _F), _f32),
    mesh=_MESH,
    compiler_params=pltpu.CompilerParams(use_tc_tiling_on_sc=False,
                                         needs_layout_passes=False),
    scratch_types=[
        pltpu.VMEM((_CROWS, _F), _f32),
        pltpu.VMEM((_CROWS, _F), _f32),
        pltpu.VMEM((_CE,), _i32),
        pltpu.VMEM((_CE,), _i32),
        pltpu.VMEM((_CE, 2), _f32),
        pltpu.VMEM((_CE, 2), _f32),
        pltpu.VMEM((_CE,), _f32),
        pltpu.VMEM((_CE,), _f32),
        pltpu.VMEM((_CROWS, _F), _f32),
        pltpu.SemaphoreType.DMA,
    ],
)
def _sc_sample(x_hbm, y_hbm, tab_hbm, out_hbm,
               xv, yv, idx0, idx1, rbuf0, rbuf1, xwv, ywv, outv, sem):
    wid = lax.axis_index("s") * _NC + lax.axis_index("c")
    row0 = wid * _ROWS_W

    lane = lax.iota(_i32, _L)
    zeros16 = jnp.zeros((_L,), _i32)
    ones16 = jnp.ones((_L,), _i32)

    def chunk_body(k, _):
        rbase = row0 + k * _CROWS
        pltpu.sync_copy(x_hbm.at[pl.ds(rbase, _CROWS)], xv)
        pltpu.sync_copy(y_hbm.at[pl.ds(rbase, _CROWS)], yv)

        def idx_row(j, _):
            for c in range(_VPR):
                sl = pl.ds(c * _L, _L)
                fl = pl.ds(j * _F + c * _L, _L)
                x = xv[j, sl] - 0.5
                y = yv[j, sl] - 0.5
                xi = x.astype(_i32)
                yi = y.astype(_i32)
                x0 = jnp.where(xi.astype(_f32) > x, xi - 1, xi)
                y0 = jnp.where(yi.astype(_f32) > y, yi - 1, yi)
                xwv[fl] = x - x0.astype(_f32)
                ywv[fl] = y - y0.astype(_f32)
                x0m = jnp.bitwise_and(x0, _RES - 1)
                y0s = jnp.bitwise_and(y0, _RES - 1) * _RP
                y1s = jnp.bitwise_and(y0 + 1, _RES - 1) * _RP
                fbase = (lane + c * _L) * _FS
                o0 = fbase + y0s + x0m
                o1 = fbase + y1s + x0m
                idx0[fl] = (lax.shift_right_logical(o0, 1)
                            + jnp.bitwise_and(o0, 1) * _HALF)
                idx1[fl] = (lax.shift_right_logical(o1, 1)
                            + jnp.bitwise_and(o1, 1) * _HALF)
            return 0

        lax.fori_loop(0, _CROWS, idx_row, 0)

        pltpu.async_copy(tab_hbm.at[idx0], rbuf0, sem)
        pltpu.async_copy(tab_hbm.at[idx1], rbuf1, sem)
        pltpu.make_async_copy(tab_hbm.at[idx0], rbuf0, sem).wait()
        pltpu.make_async_copy(tab_hbm.at[idx1], rbuf1, sem).wait()

        def lerp_row(j, _):
            for c in range(_VPR):
                sl = pl.ds(c * _L, _L)
                fl = pl.ds(j * _F + c * _L, _L)
                ev = j * _F + c * _L + lane
                a00 = plsc.load_gather(rbuf0, [ev, zeros16])
                a01 = plsc.load_gather(rbuf0, [ev, ones16])
                a10 = plsc.load_gather(rbuf1, [ev, zeros16])
                a11 = plsc.load_gather(rbuf1, [ev, ones16])
                xw = xwv[fl]
                yw = ywv[fl]
                i0 = a00 + (a01 - a00) * xw
                i1 = a10 + (a11 - a10) * xw
                outv[j, sl] = i0 + (i1 - i0) * yw
            return 0

        lax.fori_loop(0, _CROWS, lerp_row, 0)

        pltpu.sync_copy(outv, out_hbm.at[pl.ds(rbase, _CROWS)])
        return 0

    lax.fori_loop(0, _NCHUNK, chunk_body, 0)


_TROWS = _F * _FS // 128
_TBLK = 2048
_TG = _TROWS // _TBLK


def _tab_body(in_ref, out_ref):
    h = pl.program_id(0)
    a = in_ref[...]

    @pl.when(h == 0)
    def _plain():
        out_ref[...] = a

    @pl.when(h == 1)
    def _rolled():
        shift = a[:, 1:]
        col0 = a[:, :1]
        nxt = jnp.concatenate([col0[1:], col0[:1]], axis=0)
        bk3 = jnp.concatenate([col0[_TBLK - 3:], col0[:_TBLK - 3]], axis=0)
        ridx = lax.broadcasted_iota(_i32, (_TBLK, 1), 0)
        carry = jnp.where(jnp.bitwise_and(ridx, 3) == 3, bk3, nxt)
        out_ref[...] = jnp.concatenate([shift, carry], axis=1)


_build_tab = pl.pallas_call(
    _tab_body,
    grid=(2, _TG),
    in_specs=[pl.BlockSpec((_TBLK, 128), lambda h, g: (g, 0))],
    out_specs=pl.BlockSpec((_TBLK, 128), lambda h, g: (h * _TG + g, 0)),
    out_shape=jax.ShapeDtypeStruct((2 * _TROWS, 128), _f32),
)


def kernel(coords, image):
    x = coords[:, :, 0]
    y = coords[:, :, 1]
    flat2d = image.reshape(_TROWS, 128)
    tab = _build_tab(flat2d).reshape(-1, 2)
    return _sc_sample(x, y, tab)


# speedup vs baseline: 24.1242x; 22.9517x over previous
import functools

import jax
import jax.numpy as jnp
from jax import lax
from jax.experimental import pallas as pl
from jax.experimental.pallas import tpu as pltpu
from jax.experimental.pallas import tpu_sc as plsc

_F = 128
_RES = 512
_RP = _RES
_FS = _RES * _RP
_HALF8 = _F * _FS // 8
_N = 65536
_NC, _NS, _L = 2, 16, 16
_NW = _NC * _NS
_ROWS_W = _N // _NW
_CROWS = 32
_NCHUNK = _ROWS_W // _CROWS
_VPR = _F // _L
_CE = _CROWS * _F

_MESH = plsc.VectorSubcoreMesh(core_axis_name="c", subcore_axis_name="s")

_f32 = jnp.float32
_i32 = jnp.int32


@functools.partial(
    pl.kernel,
    out_type=jax.ShapeDtypeStruct((_N, _F), _f32),
    mesh=_MESH,
    compiler_params=pltpu.CompilerParams(use_tc_tiling_on_sc=False,
                                         needs_layout_passes=False),
    scratch_types=[
        pltpu.VMEM((_CROWS, _F), _f32),
        pltpu.VMEM((_CROWS, _F), _f32),
        pltpu.VMEM((_CE,), _i32),
        pltpu.VMEM((_CE,), _i32),
        pltpu.VMEM((_CE,), _i32),
        pltpu.VMEM((_CE, 8), _f32),
        pltpu.VMEM((_CE, 8), _f32),
        pltpu.VMEM((_CE,), _f32),
        pltpu.VMEM((_CE,), _f32),
        pltpu.VMEM((_CROWS, _F), _f32),
        pltpu.SemaphoreType.DMA,
    ],
)
def _sc_sample(x_hbm, y_hbm, tab_hbm, out_hbm,
               xv, yv, idx0, idx1, lanev, rbuf0, rbuf1, xwv, ywv, outv, sem):
    wid = lax.axis_index("s") * _NC + lax.axis_index("c")
    row0 = wid * _ROWS_W

    lane = lax.iota(_i32, _L)
    zeros16 = jnp.zeros((_L,), _i32)
    ones16 = jnp.ones((_L,), _i32)

    def chunk_body(k, _):
        rbase = row0 + k * _CROWS
        pltpu.sync_copy(x_hbm.at[pl.ds(rbase, _CROWS)], xv)
        pltpu.sync_copy(y_hbm.at[pl.ds(rbase, _CROWS)], yv)

        def idx_row(j, _):
            for c in range(_VPR):
                sl = pl.ds(c * _L, _L)
                fl = pl.ds(j * _F + c * _L, _L)
                x = xv[j, sl] - 0.5
                y = yv[j, sl] - 0.5
                xi = x.astype(_i32)
                yi = y.astype(_i32)
                x0 = jnp.where(xi.astype(_f32) > x, xi - 1, xi)
                y0 = jnp.where(yi.astype(_f32) > y, yi - 1, yi)
                xwv[fl] = x - x0.astype(_f32)
                ywv[fl] = y - y0.astype(_f32)
                x0m = jnp.bitwise_and(x0, _RES - 1)
                y0s = jnp.bitwise_and(y0, _RES - 1) * _RP
                y1s = jnp.bitwise_and(y0 + 1, _RES - 1) * _RP
                fbase = (lane + c * _L) * _FS
                o0 = fbase + y0s + x0m
                o1 = fbase + y1s + x0m
                lo3 = jnp.bitwise_and(o0, 7)
                ph = lax.shift_right_logical(lo3 + 1, 3)
                idx0[fl] = (lax.shift_right_logical(o0, 3) + ph * _HALF8)
                idx1[fl] = (lax.shift_right_logical(o1, 3) + ph * _HALF8)
                lanev[fl] = lo3 - ph
            return 0

        lax.fori_loop(0, _CROWS, idx_row, 0)

        pltpu.async_copy(tab_hbm.at[idx0], rbuf0, sem)
        pltpu.async_copy(tab_hbm.at[idx1], rbuf1, sem)
        pltpu.make_async_copy(tab_hbm.at[idx0], rbuf0, sem).wait()
        pltpu.make_async_copy(tab_hbm.at[idx1], rbuf1, sem).wait()

        def lerp_row(j, _):
            for c in range(_VPR):
                sl = pl.ds(c * _L, _L)
                fl = pl.ds(j * _F + c * _L, _L)
                ev = j * _F + c * _L + lane
                l00 = lanev[fl]
                l01 = l00 + 1
                a00 = plsc.load_gather(rbuf0, [ev, l00])
                a01 = plsc.load_gather(rbuf0, [ev, l01])
                a10 = plsc.load_gather(rbuf1, [ev, l00])
                a11 = plsc.load_gather(rbuf1, [ev, l01])
                xw = xwv[fl]
                yw = ywv[fl]
                i0 = a00 + (a01 - a00) * xw
                i1 = a10 + (a11 - a10) * xw
                outv[j, sl] = i0 + (i1 - i0) * yw
            return 0

        lax.fori_loop(0, _CROWS, lerp_row, 0)

        pltpu.sync_copy(outv, out_hbm.at[pl.ds(rbase, _CROWS)])
        return 0

    lax.fori_loop(0, _NCHUNK, chunk_body, 0)


_TROWS = _F * _FS // 128
_TBLK = 2048
_TG = _TROWS // _TBLK


def _tab_body(in_ref, out_ref):
    h = pl.program_id(0)
    a = in_ref[...]

    @pl.when(h == 0)
    def _plain():
        out_ref[...] = a

    @pl.when(h == 1)
    def _rolled():
        shift = a[:, 1:]
        col0 = a[:, :1]
        nxt = jnp.concatenate([col0[1:], col0[:1]], axis=0)
        bk3 = jnp.concatenate([col0[_TBLK - 3:], col0[:_TBLK - 3]], axis=0)
        ridx = lax.broadcasted_iota(_i32, (_TBLK, 1), 0)
        carry = jnp.where(jnp.bitwise_and(ridx, 3) == 3, bk3, nxt)
        out_ref[...] = jnp.concatenate([shift, carry], axis=1)


_build_tab = pl.pallas_call(
    _tab_body,
    grid=(2, _TG),
    in_specs=[pl.BlockSpec((_TBLK, 128), lambda h, g: (g, 0))],
    out_specs=pl.BlockSpec((_TBLK, 128), lambda h, g: (h * _TG + g, 0)),
    out_shape=jax.ShapeDtypeStruct((2 * _TROWS, 128), _f32),
)


def kernel(coords, image):
    x = coords[:, :, 0]
    y = coords[:, :, 1]
    flat2d = image.reshape(_TROWS, 128)
    tab = _build_tab(flat2d).reshape(-1, 8)
    return _sc_sample(x, y, tab)


# speedup vs baseline: 33.8946x; 1.4050x over previous
import functools

import jax
import jax.numpy as jnp
from jax import lax
from jax.experimental import pallas as pl
from jax.experimental.pallas import tpu as pltpu
from jax.experimental.pallas import tpu_sc as plsc

_F = 128
_RES = 512
_RP = _RES
_FS = _RES * _RP
_HALF8 = _F * _FS // 8
_N = 65536
_NC, _NS, _L = 2, 16, 16
_NW = _NC * _NS
_ROWS_W = _N // _NW
_CROWS = 16
_NCHUNK = _ROWS_W // _CROWS
_VPR = _F // _L
_CE = _CROWS * _F

_MESH = plsc.VectorSubcoreMesh(core_axis_name="c", subcore_axis_name="s")

_f32 = jnp.float32
_i32 = jnp.int32


@functools.partial(
    pl.kernel,
    out_type=jax.ShapeDtypeStruct((_N, _F), _f32),
    mesh=_MESH,
    compiler_params=pltpu.CompilerParams(use_tc_tiling_on_sc=False,
                                         needs_layout_passes=False),
    scratch_types=[
        pltpu.VMEM((_CROWS, _F), _f32),
        pltpu.VMEM((_CROWS, _F), _f32),
    ] + 2 * [
        pltpu.VMEM((_CE,), _i32),
        pltpu.VMEM((_CE,), _i32),
        pltpu.VMEM((_CE,), _i32),
        pltpu.VMEM((_CE, 8), _f32),
        pltpu.VMEM((_CE, 8), _f32),
        pltpu.VMEM((_CE,), _f32),
        pltpu.VMEM((_CE,), _f32),
        pltpu.VMEM((_CROWS, _F), _f32),
        pltpu.SemaphoreType.DMA,
    ],
)
def _sc_sample(x_hbm, y_hbm, tab_hbm, out_hbm, xv, yv,
               idx0A, idx1A, laneA, rb0A, rb1A, xwA, ywA, outA, semA,
               idx0B, idx1B, laneB, rb0B, rb1B, xwB, ywB, outB, semB):
    wid = lax.axis_index("s") * _NC + lax.axis_index("c")
    row0 = wid * _ROWS_W

    lane = lax.iota(_i32, _L)

    def fire(k, idx0, idx1, lanev, rb0, rb1, xwv, ywv, sem):
        rbase = row0 + k * _CROWS
        pltpu.sync_copy(x_hbm.at[pl.ds(rbase, _CROWS)], xv)
        pltpu.sync_copy(y_hbm.at[pl.ds(rbase, _CROWS)], yv)

        def idx_row(j, _):
            for c in range(_VPR):
                sl = pl.ds(c * _L, _L)
                fl = pl.ds(j * _F + c * _L, _L)
                x = xv[j, sl] - 0.5
                y = yv[j, sl] - 0.5
                xi = x.astype(_i32)
                yi = y.astype(_i32)
                x0 = jnp.where(xi.astype(_f32) > x, xi - 1, xi)
                y0 = jnp.where(yi.astype(_f32) > y, yi - 1, yi)
                xwv[fl] = x - x0.astype(_f32)
                ywv[fl] = y - y0.astype(_f32)
                x0m = jnp.bitwise_and(x0, _RES - 1)
                y0s = jnp.bitwise_and(y0, _RES - 1) * _RP
                y1s = jnp.bitwise_and(y0 + 1, _RES - 1) * _RP
                fbase = (lane + c * _L) * _FS
                o0 = fbase + y0s + x0m
                o1 = fbase + y1s + x0m
                lo3 = jnp.bitwise_and(o0, 7)
                ph = lax.shift_right_logical(lo3 + 1, 3)
                idx0[fl] = (lax.shift_right_logical(o0, 3) + ph * _HALF8)
                idx1[fl] = (lax.shift_right_logical(o1, 3) + ph * _HALF8)
                lanev[fl] = lo3 - ph
            return 0

        lax.fori_loop(0, _CROWS, idx_row, 0)
        pltpu.async_copy(tab_hbm.at[idx0], rb0, sem)
        pltpu.async_copy(tab_hbm.at[idx1], rb1, sem)

    def finish(k, idx0, idx1, lanev, rb0, rb1, xwv, ywv, outv, sem):
        rbase = row0 + k * _CROWS
        pltpu.make_async_copy(tab_hbm.at[idx0], rb0, sem).wait()
        pltpu.make_async_copy(tab_hbm.at[idx1], rb1, sem).wait()

        def lerp_row(j, _):
            for c in range(_VPR):
                sl = pl.ds(c * _L, _L)
                fl = pl.ds(j * _F + c * _L, _L)
                ev = j * _F + c * _L + lane
                l00 = lanev[fl]
                l01 = l00 + 1
                a00 = plsc.load_gather(rb0, [ev, l00])
                a01 = plsc.load_gather(rb0, [ev, l01])
                a10 = plsc.load_gather(rb1, [ev, l00])
                a11 = plsc.load_gather(rb1, [ev, l01])
                xw = xwv[fl]
                yw = ywv[fl]
                i0 = a00 + (a01 - a00) * xw
                i1 = a10 + (a11 - a10) * xw
                outv[j, sl] = i0 + (i1 - i0) * yw
            return 0

        lax.fori_loop(0, _CROWS, lerp_row, 0)
        pltpu.sync_copy(outv, out_hbm.at[pl.ds(rbase, _CROWS)])

    A = (idx0A, idx1A, laneA, rb0A, rb1A, xwA, ywA)
    B = (idx0B, idx1B, laneB, rb0B, rb1B, xwB, ywB)

    fire(0, *A, semA)
    fire(1, *B, semB)

    def pair_body(k2, _):
        a = 2 * k2
        finish(a, *A, outA, semA)
        fire(a + 2, *A, semA)
        finish(a + 1, *B, outB, semB)
        fire(a + 3, *B, semB)
        return 0

    lax.fori_loop(0, _NCHUNK // 2 - 1, pair_body, 0)
    finish(_NCHUNK - 2, *A, outA, semA)
    finish(_NCHUNK - 1, *B, outB, semB)


_TROWS = _F * _FS // 128
_TBLK = 2048
_TG = _TROWS // _TBLK


def _tab_body(in_ref, out_ref):
    h = pl.program_id(0)
    a = in_ref[...]

    @pl.when(h == 0)
    def _plain():
        out_ref[...] = a

    @pl.when(h == 1)
    def _rolled():
        shift = a[:, 1:]
        col0 = a[:, :1]
        nxt = jnp.concatenate([col0[1:], col0[:1]], axis=0)
        bk3 = jnp.concatenate([col0[_TBLK - 3:], col0[:_TBLK - 3]], axis=0)
        ridx = lax.broadcasted_iota(_i32, (_TBLK, 1), 0)
        carry = jnp.where(jnp.bitwise_and(ridx, 3) == 3, bk3, nxt)
        out_ref[...] = jnp.concatenate([shift, carry], axis=1)


_build_tab = pl.pallas_call(
    _tab_body,
    grid=(2, _TG),
    in_specs=[pl.BlockSpec((_TBLK, 128), lambda h, g: (g, 0))],
    out_specs=pl.BlockSpec((_TBLK, 128), lambda h, g: (h * _TG + g, 0)),
    out_shape=jax.ShapeDtypeStruct((2 * _TROWS, 128), _f32),
)


def kernel(coords, image):
    x = coords[:, :, 0]
    y = coords[:, :, 1]
    flat2d = image.reshape(_TROWS, 128)
    tab = _build_tab(flat2d).reshape(-1, 8)
    return _sc_sample(x, y, tab)
